# baseline (speedup 1.0000x reference)
import jax
import jax.numpy as jnp
from jax.experimental import pallas as pl
from jax.experimental.pallas import tpu as pltpu

BLK = 16
HT = 16
NV = 256
DIMV = 256
KPAD = 1024
NB = 8
NC = 4
IMB = 8
PAD_VAL = 1e4


def _blockify(img):
    return (img.reshape(HT, BLK, HT, BLK).transpose(0, 2, 1, 3)
            .reshape(NV, DIMV))


def _unblockify(vecs):
    return (vecs.reshape(HT, HT, BLK, BLK).transpose(0, 2, 1, 3)
            .reshape(HT * BLK, HT * BLK))


def _vq_tc_kernel(x_ref, cb_ref, rn_ref, cn_ref,
                  xhat_ref, idx_ref, loss_ref, perp_ref,
                  acc_loss, acc_counts):
    bi = pl.program_id(1)
    cb = cb_ref[0]
    flat = jnp.concatenate(
        [_blockify(x_ref[m, 0]) for m in range(IMB)], axis=0)
    rn = rn_ref[0]
    cn = cn_ref[0]
    g = jnp.dot(flat, cb.T, preferred_element_type=jnp.float32)
    d = rn - 2.0 * g + cn
    dmin = jnp.min(d, axis=1, keepdims=True)
    iota = jax.lax.broadcasted_iota(jnp.int32, d.shape, 1)
    idx = jnp.min(jnp.where(d == dmin, iota, KPAD), axis=1)
    onehot = (idx[:, None] == iota).astype(jnp.float32)
    q = jnp.dot(onehot, cb, preferred_element_type=jnp.float32)
    q_bf = q.astype(jnp.bfloat16)
    for m in range(IMB):
        xhat_ref[m, 0] = _unblockify(q_bf[m * NV:(m + 1) * NV]).astype(jnp.float32)
    idx_ref[0, 0] = idx[:, None]
    diff = q - flat
    sq = jnp.sum(diff * diff)
    counts = jnp.sum(onehot, axis=0, keepdims=True)

    @pl.when(bi == 0)
    def _init():
        acc_loss[0, 0] = sq
        acc_counts[...] = counts

    @pl.when(bi > 0)
    def _accum():
        acc_loss[0, 0] += sq
        acc_counts[...] += counts

    @pl.when(bi == NB // IMB - 1)
    def _final():
        loss_ref[...] = (1.25 / (NB * NV * DIMV) * acc_loss[0, 0]).reshape(1, 1, 1)
        avg = acc_counts[...] * (1.0 / (NB * NV))
        perp_ref[...] = jnp.exp(-jnp.sum(avg * jnp.log(avg + 1e-10))).reshape(1, 1, 1)


def kernel(x, cb0, cb1, cb2, cb3, interpret=False):
    b, c, h, w = x.shape
    ht = h // BLK
    cbs, cns = [], []
    for cb in (cb0, cb1, cb2, cb3):
        k = cb.shape[0]
        cn = (cb ** 2).sum(1)
        if k < KPAD:
            cb = jnp.concatenate(
                [cb, jnp.full((KPAD - k, DIMV), PAD_VAL, cb.dtype)], axis=0)
            cn = jnp.concatenate(
                [cn, jnp.full((KPAD - k,), PAD_VAL * PAD_VAL * DIMV, cn.dtype)])
        cbs.append(cb)
        cns.append(cn[None, :])
    cb_pad = jnp.stack(cbs, axis=0)
    cn_all = jnp.stack(cns, axis=0)

    rns = []
    for i in range(c):
        blk = x[:, i]
        vec = (blk.reshape(b, ht, BLK, ht, BLK).transpose(0, 1, 3, 2, 4)
               .reshape(b, ht * ht, BLK * BLK))
        flat = vec.reshape(-1, BLK * BLK)
        rns.append((flat ** 2).sum(1, keepdims=True))
    rn_all = jnp.stack(rns, axis=0)

    x_hat, idx, loss, perp = pl.pallas_call(
        _vq_tc_kernel,
        grid=(c, b // IMB),
        in_specs=[
            pl.BlockSpec((IMB, 1, h, w), lambda i, j: (j, i, 0, 0)),
            pl.BlockSpec((1, KPAD, DIMV), lambda i, j: (i, 0, 0)),
            pl.BlockSpec((1, IMB * NV, 1), lambda i, j: (i, j, 0)),
            pl.BlockSpec((1, 1, KPAD), lambda i, j: (i, 0, 0)),
        ],
        out_specs=[
            pl.BlockSpec((IMB, 1, h, w), lambda i, j: (j, i, 0, 0)),
            pl.BlockSpec((1, 1, IMB * NV, 1), lambda i, j: (i, j, 0, 0)),
            pl.BlockSpec((1, 1, 1), lambda i, j: (i, 0, 0)),
            pl.BlockSpec((1, 1, 1), lambda i, j: (i, 0, 0)),
        ],
        out_shape=[
            jax.ShapeDtypeStruct((b, c, h, w), jnp.float32),
            jax.ShapeDtypeStruct((c, b // IMB, IMB * NV, 1), jnp.int32),
            jax.ShapeDtypeStruct((c, 1, 1), jnp.float32),
            jax.ShapeDtypeStruct((c, 1, 1), jnp.float32),
        ],
        scratch_shapes=[
            pltpu.SMEM((1, 1), jnp.float32),
            pltpu.VMEM((1, KPAD), jnp.float32),
        ],
        interpret=interpret,
    )(x, cb_pad, rn_all, cn_all)

    indices = idx.reshape(c, b, NV).transpose(1, 0, 2)
    return (x_hat, indices, loss.reshape(c), perp.reshape(c))


# speedup vs baseline: 1.0889x; 1.0889x over previous
import jax
import jax.numpy as jnp
from jax.experimental import pallas as pl
from jax.experimental.pallas import tpu as pltpu

BLK = 16
HT = 16
NV = 256
DIMV = 256
KPAD = 1024
NB = 8
NC = 4
IMB = 8
PAD_VAL = 1e4


def _blockify(img):
    return (img.reshape(HT, BLK, HT, BLK).transpose(0, 2, 1, 3)
            .reshape(NV, DIMV))


def _unblockify(vecs):
    return (vecs.reshape(HT, HT, BLK, BLK).transpose(0, 2, 1, 3)
            .reshape(HT * BLK, HT * BLK))


def _vq_tc_kernel(x_ref, cb_ref, rn_ref, cn_ref,
                  xhat_ref, idx_ref, loss_ref, perp_ref,
                  acc_loss, acc_counts):
    bi = pl.program_id(1)
    cb = cb_ref[0]
    flat = jnp.concatenate(
        [_blockify(x_ref[m, 0]) for m in range(IMB)], axis=0)
    rn = rn_ref[0]
    cn = cn_ref[0]
    g = jnp.dot(flat, cb.T, preferred_element_type=jnp.float32)
    d = rn - 2.0 * g + cn
    dmin = jnp.min(d, axis=1, keepdims=True)
    iota = jax.lax.broadcasted_iota(jnp.int32, d.shape, 1)
    idx = jnp.min(jnp.where(d == dmin, iota, KPAD), axis=1)
    onehot = (idx[:, None] == iota).astype(jnp.float32)
    q = jnp.dot(onehot, cb, preferred_element_type=jnp.float32)
    q_bf = q.astype(jnp.bfloat16)
    for m in range(IMB):
        xhat_ref[m, 0] = _unblockify(q_bf[m * NV:(m + 1) * NV]).astype(jnp.float32)
    idx_ref[0, 0] = idx[:, None]
    diff = q - flat
    sq = jnp.sum(diff * diff)
    counts = jnp.sum(onehot, axis=0, keepdims=True)

    @pl.when(bi == 0)
    def _init():
        acc_loss[0, 0] = sq
        acc_counts[...] = counts

    @pl.when(bi > 0)
    def _accum():
        acc_loss[0, 0] += sq
        acc_counts[...] += counts

    @pl.when(bi == NB // IMB - 1)
    def _final():
        loss_ref[...] = (1.25 / (NB * NV * DIMV) * acc_loss[0, 0]).reshape(1, 1, 1)
        avg = acc_counts[...] * (1.0 / (NB * NV))
        perp_ref[...] = jnp.exp(-jnp.sum(avg * jnp.log(avg + 1e-10))).reshape(1, 1, 1)


def kernel(x, cb0, cb1, cb2, cb3, interpret=False):
    b, c, h, w = x.shape
    ht = h // BLK
    cbs, cns = [], []
    for cb in (cb0, cb1, cb2, cb3):
        k = cb.shape[0]
        cn = (cb ** 2).sum(1)
        if k < KPAD:
            cb = jnp.concatenate(
                [cb, jnp.full((KPAD - k, DIMV), PAD_VAL, cb.dtype)], axis=0)
            cn = jnp.concatenate(
                [cn, jnp.full((KPAD - k,), PAD_VAL * PAD_VAL * DIMV, cn.dtype)])
        cbs.append(cb)
        cns.append(cn[None, :])
    cb_pad = jnp.stack(cbs, axis=0)
    cn_all = jnp.stack(cns, axis=0)

    rn_all = ((x.reshape(b, c, ht, BLK, ht, BLK) ** 2).sum(axis=(3, 5))
              .transpose(1, 0, 2, 3).reshape(c, b * NV, 1))

    x_hat, idx, loss, perp = pl.pallas_call(
        _vq_tc_kernel,
        grid=(c, b // IMB),
        in_specs=[
            pl.BlockSpec((IMB, 1, h, w), lambda i, j: (j, i, 0, 0)),
            pl.BlockSpec((1, KPAD, DIMV), lambda i, j: (i, 0, 0)),
            pl.BlockSpec((1, IMB * NV, 1), lambda i, j: (i, j, 0)),
            pl.BlockSpec((1, 1, KPAD), lambda i, j: (i, 0, 0)),
        ],
        out_specs=[
            pl.BlockSpec((IMB, 1, h, w), lambda i, j: (j, i, 0, 0)),
            pl.BlockSpec((1, 1, IMB * NV, 1), lambda i, j: (i, j, 0, 0)),
            pl.BlockSpec((1, 1, 1), lambda i, j: (i, 0, 0)),
            pl.BlockSpec((1, 1, 1), lambda i, j: (i, 0, 0)),
        ],
        out_shape=[
            jax.ShapeDtypeStruct((b, c, h, w), jnp.float32),
            jax.ShapeDtypeStruct((c, b // IMB, IMB * NV, 1), jnp.int32),
            jax.ShapeDtypeStruct((c, 1, 1), jnp.float32),
            jax.ShapeDtypeStruct((c, 1, 1), jnp.float32),
        ],
        scratch_shapes=[
            pltpu.SMEM((1, 1), jnp.float32),
            pltpu.VMEM((1, KPAD), jnp.float32),
        ],
        interpret=interpret,
    )(x, cb_pad, rn_all, cn_all)

    indices = idx.reshape(c, b, NV).transpose(1, 0, 2)
    return (x_hat, indices, loss.reshape(c), perp.reshape(c))


# speedup vs baseline: 1.1022x; 1.0123x over previous
import jax
import jax.numpy as jnp
from jax.experimental import pallas as pl
from jax.experimental.pallas import tpu as pltpu

BLK = 16
HT = 16
NV = 256
DIMV = 256
KPAD = 1024
NB = 8
NC = 4
IMB = 8
PAD_VAL = 1e4


def _blockify(img):
    return (img.reshape(HT, BLK, HT, BLK).transpose(0, 2, 1, 3)
            .reshape(NV, DIMV))


def _unblockify(vecs):
    return (vecs.reshape(HT, HT, BLK, BLK).transpose(0, 2, 1, 3)
            .reshape(HT * BLK, HT * BLK))


def _vq_tc_kernel(x_ref, cb_ref, rn_ref, cn_ref,
                  xhat_ref, idx_ref, loss_ref, perp_ref,
                  acc_loss, acc_counts):
    bi = pl.program_id(1)
    cb = cb_ref[0]
    flat = jnp.concatenate(
        [_blockify(x_ref[m, 0]) for m in range(IMB)], axis=0)
    rn = rn_ref[0]
    cn = cn_ref[0]
    g = jnp.dot(flat, cb.T, preferred_element_type=jnp.float32)
    d = rn - 2.0 * g + cn
    dmin = jnp.min(d, axis=1, keepdims=True)
    iota = jax.lax.broadcasted_iota(jnp.int32, d.shape, 1)
    idx = jnp.min(jnp.where(d == dmin, iota, KPAD), axis=1)
    onehot = (idx[:, None] == iota).astype(jnp.float32)
    q = jnp.dot(onehot, cb, preferred_element_type=jnp.float32)
    q_bf = q.astype(jnp.bfloat16)
    for m in range(IMB):
        xhat_ref[m, 0] = _unblockify(q_bf[m * NV:(m + 1) * NV]).astype(jnp.float32)
    idx_ref[0, 0] = idx[:, None]
    diff = q - flat
    sq = jnp.sum(diff * diff)
    counts = jnp.sum(onehot, axis=0, keepdims=True)

    @pl.when(bi == 0)
    def _init():
        acc_loss[0, 0] = sq
        acc_counts[...] = counts

    @pl.when(bi > 0)
    def _accum():
        acc_loss[0, 0] += sq
        acc_counts[...] += counts

    @pl.when(bi == NB // IMB - 1)
    def _final():
        loss_ref[...] = (1.25 / (NB * NV * DIMV) * acc_loss[0, 0]).reshape(1, 1, 1)
        avg = acc_counts[...] * (1.0 / (NB * NV))
        perp_ref[...] = jnp.exp(-jnp.sum(avg * jnp.log(avg + 1e-10))).reshape(1, 1, 1)


def kernel(x, cb0, cb1, cb2, cb3, interpret=False):
    b, c, h, w = x.shape
    ht = h // BLK
    cbs, cns = [], []
    for cb in (cb0, cb1, cb2, cb3):
        k = cb.shape[0]
        cn = (cb ** 2).sum(1)
        if k < KPAD:
            cb = jnp.concatenate(
                [cb, jnp.full((KPAD - k, DIMV), PAD_VAL, cb.dtype)], axis=0)
            cn = jnp.concatenate(
                [cn, jnp.full((KPAD - k,), PAD_VAL * PAD_VAL * DIMV, cn.dtype)])
        cbs.append(cb)
        cns.append(cn[None, :])
    cb_pad = jnp.stack(cbs, axis=0)
    cn_all = jnp.stack(cns, axis=0)

    rn_all = ((x.reshape(b, c, ht, BLK, ht, BLK) ** 2).sum(axis=5).sum(axis=3)
              .transpose(1, 0, 2, 3).reshape(c, b * NV, 1))

    x_hat, idx, loss, perp = pl.pallas_call(
        _vq_tc_kernel,
        grid=(c, b // IMB),
        in_specs=[
            pl.BlockSpec((IMB, 1, h, w), lambda i, j: (j, i, 0, 0)),
            pl.BlockSpec((1, KPAD, DIMV), lambda i, j: (i, 0, 0)),
            pl.BlockSpec((1, IMB * NV, 1), lambda i, j: (i, j, 0)),
            pl.BlockSpec((1, 1, KPAD), lambda i, j: (i, 0, 0)),
        ],
        out_specs=[
            pl.BlockSpec((IMB, 1, h, w), lambda i, j: (j, i, 0, 0)),
            pl.BlockSpec((1, 1, IMB * NV, 1), lambda i, j: (i, j, 0, 0)),
            pl.BlockSpec((1, 1, 1), lambda i, j: (i, 0, 0)),
            pl.BlockSpec((1, 1, 1), lambda i, j: (i, 0, 0)),
        ],
        out_shape=[
            jax.ShapeDtypeStruct((b, c, h, w), jnp.float32),
            jax.ShapeDtypeStruct((c, b // IMB, IMB * NV, 1), jnp.int32),
            jax.ShapeDtypeStruct((c, 1, 1), jnp.float32),
            jax.ShapeDtypeStruct((c, 1, 1), jnp.float32),
        ],
        scratch_shapes=[
            pltpu.SMEM((1, 1), jnp.float32),
            pltpu.VMEM((1, KPAD), jnp.float32),
        ],
        interpret=interpret,
    )(x, cb_pad, rn_all, cn_all)

    indices = idx.reshape(c, b, NV).transpose(1, 0, 2)
    return (x_hat, indices, loss.reshape(c), perp.reshape(c))
